# baseline (speedup 1.0000x reference)
import jax
import jax.numpy as jnp
from jax import lax
from jax.experimental import pallas as pl
from jax.experimental.pallas import tpu as pltpu
from jax.experimental.pallas import tpu_sc as plsc

B, L, EMBED = 4096, 50, 32
NC, NS = 2, 16
NW = NC * NS
BPW = B // NW
LC = 10
NCH = L // LC
CH = LC * BPW
ET = EMBED // 8


def _emb_body(src_hbm, tgt_hbm, sidx_hbm, tidx_hbm, src_out, tgt_out,
              idx0, idx1, rows0, rows1, tbuf, si0, si1, sg0, sg1, sw):
    idx = (idx0, idx1)
    rows = (rows0, rows1)
    si = (si0, si1)
    sg = (sg0, sg1)

    wid = lax.axis_index("s") * NC + lax.axis_index("c")
    base_b = wid * BPW

    specs = []
    for ih, oh in ((sidx_hbm, src_out), (tidx_hbm, tgt_out)):
        for c in range(NCH):
            specs.append((ih, oh, c * LC))
    tabs = [src_hbm] * NCH + [tgt_hbm] * NCH
    n = len(specs)

    iota16 = lax.iota(jnp.int32, 16)
    e_splats = [iota16 * 0 + e for e in range(EMBED)]

    def idx_start(k):
        ih, _, l0 = specs[k]
        b = k % 2
        return [pltpu.async_copy(
                    ih.at[pl.ds((l0 + j) * B + base_b, BPW)],
                    idx[b].at[pl.ds(j * BPW, BPW)], si[b])
                for j in range(LC)]

    def gather_start(k):
        b = k % 2
        return pltpu.async_copy(tabs[k].at[idx[b]], rows[b], sg[b])

    def transpose_chunk(k):
        b = k % 2
        r = rows[b]

        def inner(bb, l):
            ridx = iota16 + (l * BPW + bb * 16)
            for e in range(EMBED):
                g = plsc.load_gather(r, [ridx, e_splats[e]])
                tbuf[l, e // 8, e % 8, pl.ds(bb * 16, 16)] = g
            return l

        def body(l, carry):
            lax.fori_loop(0, BPW // 16, inner, l)
            return carry

        lax.fori_loop(0, LC, body, 0)

    def write_start(k):
        _, oh, l0 = specs[k]
        return pltpu.async_copy(tbuf, oh.at[pl.ds(l0, LC), :, wid], sw)

    hi = [None] * n
    hg = [None] * n
    hw = [None] * n
    hi[0] = idx_start(0)
    hi[1] = idx_start(1)
    for k in range(n):
        for h in hi[k]:
            h.wait()
        hg[k] = gather_start(k)
        if k >= 1:
            hg[k - 1].wait()
            if k + 1 < n and k + 1 > 1:
                hi[k + 1] = idx_start(k + 1)
            if k >= 2:
                hw[k - 2].wait()
            transpose_chunk(k - 1)
            hw[k - 1] = write_start(k - 1)
    hg[n - 1].wait()
    hw[n - 2].wait()
    transpose_chunk(n - 1)
    hw[n - 1] = write_start(n - 1)
    hw[n - 1].wait()


def kernel(src_table, tgt_table, src_indices, tgt_indices):
    sidx = src_indices.T.reshape(-1).astype(jnp.int32)
    tidx = tgt_indices.T.reshape(-1).astype(jnp.int32)
    mesh = plsc.VectorSubcoreMesh(core_axis_name="c", subcore_axis_name="s")
    scratch = ([pltpu.VMEM((CH,), jnp.int32) for _ in range(2)]
               + [pltpu.VMEM((CH, EMBED), jnp.float32) for _ in range(2)]
               + [pltpu.VMEM((LC, ET, 8, 128), jnp.float32)]
               + [pltpu.SemaphoreType.DMA for _ in range(5)])
    f = pl.kernel(
        _emb_body,
        mesh=mesh,
        out_type=(
            jax.ShapeDtypeStruct((L, ET, NW, 8, 128), jnp.float32),
            jax.ShapeDtypeStruct((L, ET, NW, 8, 128), jnp.float32),
        ),
        scratch_types=scratch,
        compiler_params=pltpu.CompilerParams(use_tc_tiling_on_sc=False,
                                             needs_layout_passes=False),
    )
    src5, tgt5 = f(src_table, tgt_table, sidx, tidx)
    src_o = src5.transpose(2, 4, 0, 1, 3).reshape(B, L, EMBED)
    tgt_o = tgt5.transpose(2, 4, 0, 1, 3).reshape(B, L, EMBED)
    return (src_o, tgt_o)


# speedup vs baseline: 1.8924x; 1.8924x over previous
import jax
import jax.numpy as jnp
from jax import lax
from jax.experimental import pallas as pl
from jax.experimental.pallas import tpu as pltpu
from jax.experimental.pallas import tpu_sc as plsc

B, L, EMBED = 4096, 50, 32
NC, NS = 2, 16
NW = NC * NS
BPW = B // NW
LC = 10
NCH = L // LC
CH = LC * BPW
ET = EMBED // 8


def _emb_body(src_hbm, tgt_hbm, sidx_hbm, tidx_hbm, src_out, tgt_out,
              idx0, idx1, rows0, rows1, tbuf, si0, si1, sg0, sg1, sw):
    idx = (idx0, idx1)
    rows = (rows0, rows1)
    si = (si0, si1)
    sg = (sg0, sg1)

    wid = lax.axis_index("s") * NC + lax.axis_index("c")
    base_b = wid * BPW

    specs = []
    for ih, oh in ((sidx_hbm, src_out), (tidx_hbm, tgt_out)):
        for c in range(NCH):
            specs.append((ih, oh, c * LC))
    tabs = [src_hbm] * NCH + [tgt_hbm] * NCH
    n = len(specs)

    iota16 = lax.iota(jnp.int32, 16)

    def idx_start(k):
        ih, _, l0 = specs[k]
        b = k % 2
        return [pltpu.async_copy(
                    ih.at[pl.ds((l0 + j) * B + base_b, BPW)],
                    idx[b].at[pl.ds(j * BPW, BPW)], si[b])
                for j in range(LC)]

    def gather_start(k):
        b = k % 2
        return pltpu.async_copy(tabs[k].at[idx[b]], rows[b], sg[b])

    def transpose_chunk(k):
        b = k % 2
        r = rows[b]

        def inner(bb, l):
            ridx = iota16 + (l * BPW + bb * 16)
            bvec = iota16 + bb * 16
            lspl = iota16 * 0 + l
            for eblk in range(EMBED // 16):
                for d in range(16):
                    ce = ((iota16 + d) & 15) + eblk * 16
                    g = plsc.load_gather(r, [ridx, ce])
                    plsc.store_scatter(
                        tbuf, [lspl, ce >> 3, ce & 7, bvec], g)
            return l

        def body(l, carry):
            lax.fori_loop(0, BPW // 16, inner, l)
            return carry

        lax.fori_loop(0, LC, body, 0)

    def write_start(k):
        _, oh, l0 = specs[k]
        return pltpu.async_copy(tbuf, oh.at[pl.ds(l0, LC), :, wid], sw)

    hi = [None] * n
    hg = [None] * n
    hw = [None] * n
    hi[0] = idx_start(0)
    hi[1] = idx_start(1)
    for k in range(n):
        for h in hi[k]:
            h.wait()
        hg[k] = gather_start(k)
        if k >= 1:
            hg[k - 1].wait()
            if k + 1 < n and k + 1 > 1:
                hi[k + 1] = idx_start(k + 1)
            if k >= 2:
                hw[k - 2].wait()
            transpose_chunk(k - 1)
            hw[k - 1] = write_start(k - 1)
    hg[n - 1].wait()
    hw[n - 2].wait()
    transpose_chunk(n - 1)
    hw[n - 1] = write_start(n - 1)
    hw[n - 1].wait()


def kernel(src_table, tgt_table, src_indices, tgt_indices):
    sidx = src_indices.T.reshape(-1).astype(jnp.int32)
    tidx = tgt_indices.T.reshape(-1).astype(jnp.int32)
    mesh = plsc.VectorSubcoreMesh(core_axis_name="c", subcore_axis_name="s")
    scratch = ([pltpu.VMEM((CH,), jnp.int32) for _ in range(2)]
               + [pltpu.VMEM((CH, EMBED), jnp.float32) for _ in range(2)]
               + [pltpu.VMEM((LC, ET, 8, 128), jnp.float32)]
               + [pltpu.SemaphoreType.DMA for _ in range(5)])
    f = pl.kernel(
        _emb_body,
        mesh=mesh,
        out_type=(
            jax.ShapeDtypeStruct((L, ET, NW, 8, 128), jnp.float32),
            jax.ShapeDtypeStruct((L, ET, NW, 8, 128), jnp.float32),
        ),
        scratch_types=scratch,
        compiler_params=pltpu.CompilerParams(use_tc_tiling_on_sc=False,
                                             needs_layout_passes=False),
    )
    src5, tgt5 = f(src_table, tgt_table, sidx, tidx)
    src_o = src5.transpose(2, 4, 0, 1, 3).reshape(B, L, EMBED)
    tgt_o = tgt5.transpose(2, 4, 0, 1, 3).reshape(B, L, EMBED)
    return (src_o, tgt_o)
